# baseline (speedup 1.0000x reference)
import jax
import jax.numpy as jnp
from jax import lax
from jax.experimental import pallas as pl
from jax.experimental.pallas import tpu as pltpu
from jax.experimental.pallas import tpu_sc as plsc

D = 128
NC = 2
NS = 16
NW = NC * NS
CHUNK = 80
NBUF = 4
UNROLL = 8


def _permute_body(x_hbm, perm_hbm, out_hbm, perm_v, *rest):
    in_bufs = rest[:NBUF]
    out_bufs = rest[NBUF:2 * NBUF]
    in_sems = rest[2 * NBUF:3 * NBUF]
    out_sems = rest[3 * NBUF:4 * NBUF]

    rows = x_hbm.shape[0]
    rows_per_w = rows // NW
    nchunk = rows_per_w // CHUNK
    nround = nchunk // NBUF
    wid = lax.axis_index("s") * NC + lax.axis_index("c")
    base = wid * rows_per_w

    pltpu.sync_copy(perm_hbm, perm_v)
    pvecs = [perm_v[pl.ds(16 * j, 16)] for j in range(D // 16)]

    def compute(in_v, out_v):
        @plsc.parallel_loop(0, CHUNK, unroll=UNROLL)
        def _(r):
            rs = jnp.full((16,), r, jnp.int32)
            for j in range(D // 16):
                v = plsc.load_gather(in_v, [rs, pvecs[j]])
                out_v[r, pl.ds(16 * j, 16)] = v

    def copy_in(c, in_v, si):
        return pltpu.make_async_copy(
            x_hbm.at[pl.ds(base + c * CHUNK, CHUNK)], in_v, si)

    def copy_out(c, out_v, so):
        return pltpu.make_async_copy(
            out_v, out_hbm.at[pl.ds(base + c * CHUNK, CHUNK)], so)

    for k in range(NBUF):
        copy_in(k, in_bufs[k], in_sems[k]).start()

    def loop_body(rd, carry):
        for k in range(NBUF):
            c = rd * NBUF + k
            copy_in(c, in_bufs[k], in_sems[k]).wait()

            @pl.when(rd > 0)
            def _():
                copy_out(c - NBUF, out_bufs[k], out_sems[k]).wait()

            compute(in_bufs[k], out_bufs[k])
            copy_out(c, out_bufs[k], out_sems[k]).start()

            @pl.when(rd < nround - 1)
            def _():
                copy_in(c + NBUF, in_bufs[k], in_sems[k]).start()
        return carry

    lax.fori_loop(0, nround, loop_body, 0)
    for k in range(NBUF):
        copy_out(nchunk - NBUF + k, out_bufs[k], out_sems[k]).wait()


def kernel(x, permutation):
    b, s, d = x.shape
    rows = b * s
    xt = jnp.transpose(x, (1, 0, 2)).reshape(rows, d)
    perm = permutation.astype(jnp.int32)

    mesh = plsc.VectorSubcoreMesh(core_axis_name="c", subcore_axis_name="s")
    run = pl.kernel(
        _permute_body,
        out_type=jax.ShapeDtypeStruct((rows, d), jnp.float32),
        mesh=mesh,
        scratch_types=(
            [pltpu.VMEM((D,), jnp.int32)]
            + [pltpu.VMEM((CHUNK, D), jnp.float32)] * (2 * NBUF)
            + [pltpu.SemaphoreType.DMA] * (2 * NBUF)
        ),
        compiler_params=pltpu.CompilerParams(
            needs_layout_passes=False, use_tc_tiling_on_sc=True),
    )
    out = run(xt, perm)
    return jnp.transpose(out.reshape(s, b, d), (1, 0, 2))
